# Initial kernel scaffold; baseline (speedup 1.0000x reference)
import jax
import jax.numpy as jnp
from jax.experimental import pallas as pl


def kernel(features, idx):
    raise NotImplementedError("write your pallas kernel here")


# speedup vs baseline: 2.0020x; 2.0020x over previous
import functools

import jax
import jax.numpy as jnp
from jax import lax
from jax.experimental import pallas as pl
from jax.experimental.pallas import tpu as pltpu
from jax.experimental.pallas import tpu_sc as plsc

_B, _C, _N = 8, 256, 16384
_M = 4096
_L = 16
_NC, _NS = 2, 16
_NW = _NC * _NS
_R = _B * _C
_RPW = _R // _NW


@functools.partial(
    pl.kernel,
    out_type=jax.ShapeDtypeStruct((_R, _M), jnp.float32),
    mesh=plsc.VectorSubcoreMesh(core_axis_name="c", subcore_axis_name="s"),
    compiler_params=pltpu.CompilerParams(needs_layout_passes=False),
    scratch_types=[
        pltpu.VMEM((_M,), jnp.int32),
        pltpu.VMEM((_N,), jnp.float32),
        pltpu.VMEM((_M,), jnp.float32),
    ],
)
def _gather_rows(feat_hbm, idx_hbm, out_hbm, idx_v, feat_v, out_v):
    wid = lax.axis_index("s") * _NC + lax.axis_index("c")
    base = wid * _RPW
    b = base // _C
    pltpu.sync_copy(idx_hbm.at[b], idx_v)

    def row_body(i, carry):
        r = base + i
        pltpu.sync_copy(feat_hbm.at[r], feat_v)

        def j_body(j, c2):
            iv = idx_v[pl.ds(j * _L, _L)]
            out_v[pl.ds(j * _L, _L)] = plsc.load_gather(feat_v, [iv])
            return c2

        lax.fori_loop(0, _M // _L, j_body, 0)
        pltpu.sync_copy(out_v, out_hbm.at[r])
        return carry

    lax.fori_loop(0, _RPW, row_body, 0)


def kernel(features, idx):
    feat2d = features.reshape(_R, _N)
    out2d = _gather_rows(feat2d, idx)
    return out2d.reshape(_B, _C, _M)


# speedup vs baseline: 5.5793x; 2.7869x over previous
import functools

import jax
import jax.numpy as jnp
from jax import lax
from jax.experimental import pallas as pl
from jax.experimental.pallas import tpu as pltpu
from jax.experimental.pallas import tpu_sc as plsc

_B, _C, _N = 8, 256, 16384
_M = 4096
_L = 16
_NC, _NS = 2, 16
_NW = _NC * _NS
_R = _B * _C
_RPW = _R // _NW
_NBUF = 3


@functools.partial(
    pl.kernel,
    out_type=jax.ShapeDtypeStruct((_R, _M), jnp.float32),
    mesh=plsc.VectorSubcoreMesh(core_axis_name="c", subcore_axis_name="s"),
    compiler_params=pltpu.CompilerParams(needs_layout_passes=False),
    scratch_types=[
        pltpu.VMEM((_M,), jnp.int32),
        pltpu.VMEM((_N,), jnp.float32),
        pltpu.VMEM((_N,), jnp.float32),
        pltpu.VMEM((_N,), jnp.float32),
        pltpu.VMEM((_M,), jnp.float32),
        pltpu.VMEM((_M,), jnp.float32),
        pltpu.VMEM((_M,), jnp.float32),
        pltpu.SemaphoreType.DMA,
        pltpu.SemaphoreType.DMA,
        pltpu.SemaphoreType.DMA,
        pltpu.SemaphoreType.DMA,
        pltpu.SemaphoreType.DMA,
        pltpu.SemaphoreType.DMA,
    ],
)
def _gather_rows(feat_hbm, idx_hbm, out_hbm, idx_v, fv0, fv1, fv2,
                 ov0, ov1, ov2, fs0, fs1, fs2, os0, os1, os2):
    fv = (fv0, fv1, fv2)
    ov = (ov0, ov1, ov2)
    fsem = (fs0, fs1, fs2)
    osem = (os0, os1, os2)
    wid = lax.axis_index("s") * _NC + lax.axis_index("c")
    base = wid * _RPW
    pltpu.sync_copy(idx_hbm.at[base // _C], idx_v)

    for k in range(_NBUF):
        pltpu.async_copy(feat_hbm.at[base + k], fv[k], fsem[k])

    def group_body(g, carry):
        i = g * _NBUF
        for k in range(_NBUF):
            r = base + i + k
            pltpu.make_async_copy(feat_hbm.at[r], fv[k], fsem[k]).wait()

            @pl.when(i + k >= _NBUF)
            def _wait_out():
                pltpu.make_async_copy(ov[k], out_hbm.at[r], osem[k]).wait()

            @plsc.parallel_loop(0, _M, step=_L, unroll=8)
            def _gather(j):
                iv = idx_v[pl.ds(j, _L)]
                ov[k][pl.ds(j, _L)] = plsc.load_gather(fv[k], [iv])

            pltpu.async_copy(ov[k], out_hbm.at[r], osem[k])

            @pl.when(i + k + _NBUF < _RPW)
            def _prefetch():
                pltpu.async_copy(feat_hbm.at[r + _NBUF], fv[k], fsem[k])
        return carry

    lax.fori_loop(0, _RPW // _NBUF, group_body, 0)

    rem = _RPW - (_RPW // _NBUF) * _NBUF
    for k in range(rem):
        r = base + _RPW - rem + k
        pltpu.make_async_copy(feat_hbm.at[r], fv[k], fsem[k]).wait()
        pltpu.make_async_copy(ov[k], out_hbm.at[r], osem[k]).wait()

        @plsc.parallel_loop(0, _M, step=_L, unroll=8)
        def _gather_tail(j):
            iv = idx_v[pl.ds(j, _L)]
            ov[k][pl.ds(j, _L)] = plsc.load_gather(fv[k], [iv])

        pltpu.async_copy(ov[k], out_hbm.at[r], osem[k])

    for k in range(_NBUF):
        pltpu.make_async_copy(ov[k], out_hbm.at[base], osem[k]).wait()


def kernel(features, idx):
    feat2d = features.reshape(_R, _N)
    out2d = _gather_rows(feat2d, idx)
    return out2d.reshape(_B, _C, _M)


# speedup vs baseline: 5.7409x; 1.0290x over previous
import functools

import jax
import jax.numpy as jnp
from jax import lax
from jax.experimental import pallas as pl
from jax.experimental.pallas import tpu as pltpu
from jax.experimental.pallas import tpu_sc as plsc

_B, _C, _N = 8, 256, 16384
_M = 4096
_L = 16
_NC, _NS = 2, 16
_NW = _NC * _NS
_R = _B * _C
_RPW = _R // _NW
_NBUF = 4


@functools.partial(
    pl.kernel,
    out_type=jax.ShapeDtypeStruct((_R, _M), jnp.float32),
    mesh=plsc.VectorSubcoreMesh(core_axis_name="c", subcore_axis_name="s"),
    compiler_params=pltpu.CompilerParams(needs_layout_passes=False),
    scratch_types=[
        pltpu.VMEM((_M,), jnp.int32),
        pltpu.VMEM((_N,), jnp.float32),
        pltpu.VMEM((_N,), jnp.float32),
        pltpu.VMEM((_N,), jnp.float32),
        pltpu.VMEM((_N,), jnp.float32),
        pltpu.VMEM((_M,), jnp.float32),
        pltpu.VMEM((_M,), jnp.float32),
        pltpu.VMEM((_M,), jnp.float32),
        pltpu.VMEM((_M,), jnp.float32),
        pltpu.SemaphoreType.DMA,
        pltpu.SemaphoreType.DMA,
        pltpu.SemaphoreType.DMA,
        pltpu.SemaphoreType.DMA,
        pltpu.SemaphoreType.DMA,
        pltpu.SemaphoreType.DMA,
        pltpu.SemaphoreType.DMA,
        pltpu.SemaphoreType.DMA,
    ],
)
def _gather_rows(feat_hbm, idx_hbm, out_hbm, idx_v, fv0, fv1, fv2, fv3,
                 ov0, ov1, ov2, ov3, fs0, fs1, fs2, fs3, os0, os1, os2, os3):
    fv = (fv0, fv1, fv2, fv3)
    ov = (ov0, ov1, ov2, ov3)
    fsem = (fs0, fs1, fs2, fs3)
    osem = (os0, os1, os2, os3)
    wid = lax.axis_index("s") * _NC + lax.axis_index("c")
    base = wid * _RPW
    pltpu.sync_copy(idx_hbm.at[base // _C], idx_v)

    for k in range(_NBUF):
        pltpu.async_copy(feat_hbm.at[base + k], fv[k], fsem[k])

    def group_body(g, carry):
        i = g * _NBUF
        for k in range(_NBUF):
            r = base + i + k
            pltpu.make_async_copy(feat_hbm.at[r], fv[k], fsem[k]).wait()

            @pl.when(i + k >= _NBUF)
            def _wait_out():
                pltpu.make_async_copy(ov[k], out_hbm.at[r], osem[k]).wait()

            @plsc.parallel_loop(0, _M, step=_L, unroll=8)
            def _gather(j):
                iv = idx_v[pl.ds(j, _L)]
                ov[k][pl.ds(j, _L)] = plsc.load_gather(fv[k], [iv])

            pltpu.async_copy(ov[k], out_hbm.at[r], osem[k])

            @pl.when(i + k + _NBUF < _RPW)
            def _prefetch():
                pltpu.async_copy(feat_hbm.at[r + _NBUF], fv[k], fsem[k])
        return carry

    lax.fori_loop(0, _RPW // _NBUF, group_body, 0)

    rem = _RPW - (_RPW // _NBUF) * _NBUF
    for k in range(rem):
        r = base + _RPW - rem + k
        pltpu.make_async_copy(feat_hbm.at[r], fv[k], fsem[k]).wait()
        pltpu.make_async_copy(ov[k], out_hbm.at[r], osem[k]).wait()

        @plsc.parallel_loop(0, _M, step=_L, unroll=8)
        def _gather_tail(j):
            iv = idx_v[pl.ds(j, _L)]
            ov[k][pl.ds(j, _L)] = plsc.load_gather(fv[k], [iv])

        pltpu.async_copy(ov[k], out_hbm.at[r], osem[k])

    for k in range(_NBUF):
        pltpu.make_async_copy(ov[k], out_hbm.at[base], osem[k]).wait()


def kernel(features, idx):
    feat2d = features.reshape(_R, _N)
    out2d = _gather_rows(feat2d, idx)
    return out2d.reshape(_B, _C, _M)
